# Initial kernel scaffold; baseline (speedup 1.0000x reference)
import jax
import jax.numpy as jnp
from jax.experimental import pallas as pl


def kernel(x, edge_index, W1, att_src1, att_dst1, b1, W2, att_src2, att_dst2, b2):
    raise NotImplementedError("write your pallas kernel here")


# speedup vs baseline: 128.5286x; 128.5286x over previous
import jax
import jax.numpy as jnp
from jax import lax
from jax.experimental import pallas as pl
from jax.experimental.pallas import tpu as pltpu
from jax.experimental.pallas import tpu_sc as plsc


N = 10000
NP = 10240
DF = 128
D1 = 64
NC2 = 40
E_TOT = 320000 + N
NWORK = 32
EGRP = 128
GPW = 81
E_PAD = NWORK * GPW * EGRP
ROW1 = 80
ROW2 = 48
RPT = NP // 16
BLK = 256
GRID = NP // BLK


def _tc1_body(x_ref, w1_ref, ms_ref, md_ref, st_ref, dt_ref):
    h = jnp.dot(x_ref[...], w1_ref[...], preferred_element_type=jnp.float32)
    asrc = jnp.dot(h, ms_ref[...], preferred_element_type=jnp.float32)
    adst = jnp.dot(h, md_ref[...], preferred_element_type=jnp.float32)
    ones = jnp.ones((BLK, 8), jnp.float32)
    zeros = jnp.zeros((BLK, 8), jnp.float32)
    st_ref[...] = jnp.concatenate([h, asrc, ones], axis=1)
    dt_ref[...] = jnp.concatenate([adst, zeros], axis=1)


def _tc2_body(parts_ref, b1_ref, w2_ref, as2_ref, ad2_ref, r_ref,
              st_ref, as_ref, ad_ref):
    p = parts_ref[0] + parts_ref[1]
    num = p[:, 0:D1]
    den = p[:, 72:80] + 1e-16
    den_b = jnp.dot(den, r_ref[...], preferred_element_type=jnp.float32)
    h1 = num / den_b + b1_ref[...]
    h1 = jnp.where(h1 > 0, h1, jnp.exp(h1) - 1.0)
    h2 = jnp.dot(h1, w2_ref[...], preferred_element_type=jnp.float32)
    a2s = jnp.dot(h2, as2_ref[...], preferred_element_type=jnp.float32)
    a2d = jnp.dot(h2, ad2_ref[...], preferred_element_type=jnp.float32)
    onec = jnp.ones((BLK, 1), jnp.float32)
    pad7 = jnp.zeros((BLK, 7), jnp.float32)
    st_ref[...] = jnp.concatenate([h2, onec, pad7], axis=1)
    as_ref[...] = a2s
    ad_ref[...] = a2d


def _tc3_body(parts_ref, b2_ref, out_ref):
    q = parts_ref[0] + parts_ref[1]
    logits = q[:, 0:NC2] / (q[:, 40:41] + 1e-16) + b2_ref[...]
    m = jnp.max(logits, axis=1, keepdims=True)
    lse = m + jnp.log(jnp.sum(jnp.exp(logits - m), axis=1, keepdims=True))
    out_ref[...] = logits - lse


def _bcast_idx(lane, spec):
    if spec[0] == "pair":
        return jnp.right_shift(lane, 3) + spec[1]
    if spec[0] == "mod":
        return jnp.bitwise_and(lane, 7)
    return jnp.bitwise_and(lane, 0) + spec[1]


def _sc_pipe(row_w, c, s, w, st_hbm, sidx_hbm, didx_hbm, out_hbm,
             sidx, didx, S, MSG, acc, gsA, gsB, isem, ssem0, ssem1,
             start_extra, wait_extra, compute):
    nv = row_w // 16
    zero16 = jnp.zeros((16,), jnp.float32)

    @plsc.parallel_loop(0, EGRP, unroll=4)
    def _zb(i):
        for j in range(nv):
            MSG[0, i, pl.ds(16 * j, 16)] = zero16
    for k in range(RPT // EGRP):
        pltpu.sync_copy(MSG.at[0], acc.at[pl.ds(s * RPT + k * EGRP, EGRP)])
    plsc.subcore_barrier()

    def _start_gathers(p5, p3, m):
        @pl.when(m == 0)
        def _sg0():
            pltpu.async_copy(st_hbm.at[sidx.at[p5]], S.at[p3], gsA)

        @pl.when(m == 1)
        def _sg1():
            pltpu.async_copy(st_hbm.at[sidx.at[p5]], S.at[p3], gsB)
        start_extra(p5, p3, m)

    def _wait_gathers(p5, p3, m):
        @pl.when(m == 0)
        def _wg0():
            pltpu.make_async_copy(st_hbm.at[sidx.at[p5]], S.at[p3], gsA).wait()

        @pl.when(m == 1)
        def _wg1():
            pltpu.make_async_copy(st_hbm.at[sidx.at[p5]], S.at[p3], gsB).wait()
        wait_extra(p5, p3, m)

    for g0 in (0, 1):
        pltpu.sync_copy(sidx_hbm.at[w, g0], sidx.at[g0])
        pltpu.sync_copy(didx_hbm.at[w, g0], didx.at[g0])
        _start_gathers(g0, g0, jnp.int32(g0))
    pltpu.async_copy(sidx_hbm.at[w, 2], sidx.at[2], isem)
    pltpu.async_copy(didx_hbm.at[w, 2], didx.at[2], isem)

    def grp(g, carry):
        p5 = lax.rem(g, 5)
        p3 = lax.rem(g, 3)
        m = jnp.bitwise_and(g, 1)

        @pl.when((g >= 2) & (m == 0))
        def _wait_scatter0():
            pltpu.make_async_copy(
                MSG.at[0], acc.at[didx.at[p5]], ssem0).wait()

        @pl.when((g >= 2) & (m == 1))
        def _wait_scatter1():
            pltpu.make_async_copy(
                MSG.at[1], acc.at[didx.at[p5]], ssem1).wait()

        _wait_gathers(p5, p3, m)

        @pl.when(g + 2 < GPW)
        def _launch_ahead():
            q5 = lax.rem(g + 2, 5)
            q3 = lax.rem(g + 2, 3)
            pltpu.make_async_copy(sidx_hbm.at[w, g + 2], sidx.at[q5], isem).wait()
            pltpu.make_async_copy(didx_hbm.at[w, g + 2], didx.at[q5], isem).wait()
            _start_gathers(q5, q3, m)

        @pl.when(g + 3 < GPW)
        def _idx_ahead():
            r5 = lax.rem(g + 3, 5)
            pltpu.async_copy(sidx_hbm.at[w, g + 3], sidx.at[r5], isem)
            pltpu.async_copy(didx_hbm.at[w, g + 3], didx.at[r5], isem)

        compute(p3, m)

        @pl.when(m == 0)
        def _scat0():
            pltpu.async_copy(MSG.at[0], acc.at[didx.at[p5]], ssem0, add=True)

        @pl.when(m == 1)
        def _scat1():
            pltpu.async_copy(MSG.at[1], acc.at[didx.at[p5]], ssem1, add=True)
        return carry
    lax.fori_loop(0, GPW, grp, 0)

    for g, sem_e in ((GPW - 2, ssem1), (GPW - 1, ssem0)):
        pltpu.make_async_copy(
            MSG.at[g & 1], acc.at[didx.at[g % 5]], sem_e).wait()
    plsc.subcore_barrier()
    for k in range(RPT // EGRP):
        rows = pl.ds(s * RPT + k * EGRP, EGRP)
        pltpu.sync_copy(acc.at[rows], MSG.at[0])
        pltpu.sync_copy(MSG.at[0], out_hbm.at[c, rows])


def _make_sc1():
    row_w = ROW1

    def body(st_hbm, dt_hbm, sidx_hbm, didx_hbm, out_hbm,
             sidx, didx, S, D, MSG, TBA, acc, gsA, gsB, gs2A, gs2B, isem,
             ssem0, ssem1):
        c = lax.axis_index("c")
        s = lax.axis_index("s")
        w = c * 16 + s

        def start_extra(p5, p3, m):
            @pl.when(m == 0)
            def _se0():
                pltpu.async_copy(dt_hbm.at[didx.at[p5]], D.at[p3], gs2A)

            @pl.when(m == 1)
            def _se1():
                pltpu.async_copy(dt_hbm.at[didx.at[p5]], D.at[p3], gs2B)

        def wait_extra(p5, p3, m):
            @pl.when(m == 0)
            def _we0():
                pltpu.make_async_copy(
                    dt_hbm.at[didx.at[p5]], D.at[p3], gs2A).wait()

            @pl.when(m == 1)
            def _we1():
                pltpu.make_async_copy(
                    dt_hbm.at[didx.at[p5]], D.at[p3], gs2B).wait()

        def compute(p, m):
            @plsc.parallel_loop(0, EGRP, unroll=4)
            def _edge(i):
                lane = lax.broadcasted_iota(jnp.int32, (16,), 0)
                u = S[p, i, pl.ds(64, 16)] + D[p, i, :]
                u = jnp.where(u >= 0.0, u, 0.2 * u)
                TBA[i, :] = jnp.exp(u)
                irow = jnp.bitwise_and(lane, 0) + i
                for j, spec in enumerate([("pair", 0), ("pair", 2),
                                          ("pair", 4), ("pair", 6)]):
                    sb = plsc.load_gather(TBA, [irow, _bcast_idx(lane, spec)])
                    MSG[m, i, pl.ds(16 * j, 16)] = (
                        S[p, i, pl.ds(16 * j, 16)] * sb)
                sb4 = plsc.load_gather(TBA, [irow, _bcast_idx(lane, ("mod",))])
                MSG[m, i, pl.ds(64, 16)] = sb4

        _sc_pipe(row_w, c, s, w, st_hbm, sidx_hbm, didx_hbm, out_hbm,
                 sidx, didx, S, MSG, acc, gsA, gsB, isem, ssem0, ssem1,
                 start_extra, wait_extra, compute)

    mesh = plsc.VectorSubcoreMesh(core_axis_name="c", subcore_axis_name="s",
                                  num_cores=2, num_subcores=16)
    return pl.kernel(
        body,
        out_type=jax.ShapeDtypeStruct((2, NP, row_w), jnp.float32),
        mesh=mesh,
        compiler_params=pltpu.CompilerParams(use_tc_tiling_on_sc=False,
                                             needs_layout_passes=False),
        scratch_types=[
            pltpu.VMEM((5, EGRP), jnp.int32),
            pltpu.VMEM((5, EGRP), jnp.int32),
            pltpu.VMEM((3, EGRP, row_w), jnp.float32),
            pltpu.VMEM((3, EGRP, 16), jnp.float32),
            pltpu.VMEM((2, EGRP, row_w), jnp.float32),
            pltpu.VMEM((EGRP, 16), jnp.float32),
            pltpu.VMEM_SHARED((NP, row_w), jnp.float32),
            pltpu.SemaphoreType.DMA,
            pltpu.SemaphoreType.DMA,
            pltpu.SemaphoreType.DMA,
            pltpu.SemaphoreType.DMA,
            pltpu.SemaphoreType.DMA,
            pltpu.SemaphoreType.DMA,
            pltpu.SemaphoreType.DMA,
        ],
    )


def _make_sc2():
    row_w = ROW2

    def body(st_hbm, as_hbm, ad_hbm, sidx_hbm, didx_hbm, out_hbm,
             sidx, didx, S, AS, AD, SBUF, MSG, acc, gsA, gsB, isem, ssem0,
             ssem1):
        c = lax.axis_index("c")
        s = lax.axis_index("s")
        w = c * 16 + s
        pltpu.sync_copy(as_hbm, AS)
        pltpu.sync_copy(ad_hbm, AD)

        def start_extra(p5, p3, m):
            pass

        def wait_extra(p5, p3, m):
            pass

        def compute(p, m):
            @plsc.parallel_loop(0, EGRP // 16, unroll=2)
            def _att(k):
                sv = sidx[p, pl.ds(16 * k, 16)]
                dv = didx[p, pl.ds(16 * k, 16)]
                a = plsc.load_gather(AS, [sv]) + plsc.load_gather(AD, [dv])
                a = jnp.where(a >= 0.0, a, 0.2 * a)
                SBUF[pl.ds(16 * k, 16)] = jnp.exp(a)

            @plsc.parallel_loop(0, EGRP, unroll=4)
            def _edge(i):
                lane = lax.broadcasted_iota(jnp.int32, (16,), 0)
                sb = plsc.load_gather(SBUF, [jnp.bitwise_and(lane, 0) + i])
                for j in range(row_w // 16):
                    MSG[m, i, pl.ds(16 * j, 16)] = (
                        S[p, i, pl.ds(16 * j, 16)] * sb)

        _sc_pipe(row_w, c, s, w, st_hbm, sidx_hbm, didx_hbm, out_hbm,
                 sidx, didx, S, MSG, acc, gsA, gsB, isem, ssem0, ssem1,
                 start_extra, wait_extra, compute)

    mesh = plsc.VectorSubcoreMesh(core_axis_name="c", subcore_axis_name="s",
                                  num_cores=2, num_subcores=16)
    return pl.kernel(
        body,
        out_type=jax.ShapeDtypeStruct((2, NP, row_w), jnp.float32),
        mesh=mesh,
        compiler_params=pltpu.CompilerParams(use_tc_tiling_on_sc=False,
                                             needs_layout_passes=False),
        scratch_types=[
            pltpu.VMEM((5, EGRP), jnp.int32),
            pltpu.VMEM((5, EGRP), jnp.int32),
            pltpu.VMEM((3, EGRP, row_w), jnp.float32),
            pltpu.VMEM((NP,), jnp.float32),
            pltpu.VMEM((NP,), jnp.float32),
            pltpu.VMEM((EGRP,), jnp.float32),
            pltpu.VMEM((2, EGRP, row_w), jnp.float32),
            pltpu.VMEM_SHARED((NP, row_w), jnp.float32),
            pltpu.SemaphoreType.DMA,
            pltpu.SemaphoreType.DMA,
            pltpu.SemaphoreType.DMA,
            pltpu.SemaphoreType.DMA,
            pltpu.SemaphoreType.DMA,
        ],
    )


def kernel(x, edge_index, W1, att_src1, att_dst1, b1, W2, att_src2, att_dst2, b2):
    f32 = jnp.float32
    loop = jnp.arange(N, dtype=jnp.int32)
    padi = jnp.full((E_PAD - E_TOT,), N, jnp.int32)
    src = jnp.concatenate([edge_index[0].astype(jnp.int32), loop, padi])
    dst = jnp.concatenate([edge_index[1].astype(jnp.int32), loop, padi])
    src3 = src.reshape(NWORK, GPW, EGRP)
    dst3 = dst.reshape(NWORK, GPW, EGRP)

    eye8 = jnp.eye(8, dtype=f32)
    m1s = (att_src1[:, :, None] * eye8[:, None, :]).reshape(D1, 8)
    m1d = (att_dst1[:, :, None] * eye8[:, None, :]).reshape(D1, 8)
    rmat = jnp.kron(eye8, jnp.ones((1, 8), f32))

    full = lambda shp: pl.BlockSpec(shp, lambda i: (0, 0))
    rows = lambda wdt: pl.BlockSpec((BLK, wdt), lambda i: (i, 0))

    tc1 = pl.pallas_call(
        _tc1_body,
        grid=(GRID,),
        in_specs=[rows(DF), full((DF, D1)), full((D1, 8)), full((D1, 8))],
        out_specs=[rows(ROW1), rows(16)],
        out_shape=[jax.ShapeDtypeStruct((NP, ROW1), f32),
                   jax.ShapeDtypeStruct((NP, 16), f32)],
    )
    st1, dt1 = tc1(x, W1, m1s, m1d)

    sc1 = _make_sc1()
    parts1 = sc1(st1, dt1, src3, dst3)

    tc2 = pl.pallas_call(
        _tc2_body,
        grid=(GRID,),
        in_specs=[pl.BlockSpec((2, BLK, ROW1), lambda i: (0, i, 0)),
                  full((1, D1)), full((D1, NC2)),
                  full((NC2, 1)), full((NC2, 1)), full((8, D1))],
        out_specs=[rows(ROW2), rows(1), rows(1)],
        out_shape=[jax.ShapeDtypeStruct((NP, ROW2), f32),
                   jax.ShapeDtypeStruct((NP, 1), f32),
                   jax.ShapeDtypeStruct((NP, 1), f32)],
    )
    st2, a2s, a2d = tc2(parts1, b1.reshape(1, D1), W2,
                        att_src2.reshape(NC2, 1), att_dst2.reshape(NC2, 1),
                        rmat)

    sc2 = _make_sc2()
    parts2 = sc2(st2, a2s.reshape(NP), a2d.reshape(NP), src3, dst3)

    tc3 = pl.pallas_call(
        _tc3_body,
        grid=(GRID,),
        in_specs=[pl.BlockSpec((2, BLK, ROW2), lambda i: (0, i, 0)),
                  full((1, NC2))],
        out_specs=[rows(NC2)],
        out_shape=[jax.ShapeDtypeStruct((N, NC2), f32)],
    )
    return tc3(parts2, b2.reshape(1, NC2))[0]
